# Initial kernel scaffold; baseline (speedup 1.0000x reference)
import jax
import jax.numpy as jnp
from jax.experimental import pallas as pl


def kernel(x, edge_index, W_in, b_in, W0, W1, W_cls, b_cls):
    raise NotImplementedError("write your pallas kernel here")


# speedup vs baseline: 1.8442x; 1.8442x over previous
import jax
import jax.numpy as jnp
from jax import lax
from jax.experimental import pallas as pl
from jax.experimental.pallas import tpu as pltpu
from jax.experimental.pallas import tpu_sc as plsc

N = 10000
E = 320000
D = 128
C = 40

NC = 2
NS = 16
NW = NC * NS

N_PAD = 10240
NODES_PER_TILE = N_PAD // NS
ZB = 80

K = 80
E_PAD = 327680
CHUNKS_PER_TILE = E_PAD // (NW * K)

_MESH = plsc.VectorSubcoreMesh(
    core_axis_name="c", subcore_axis_name="s", num_cores=NC, num_subcores=NS)


def _sc_degree_body(col_hbm, ones_hbm, zeros_hbm, out_hbm,
                    cidx, onesbuf, dacc, sem):
  cid = lax.axis_index("c")
  sid = lax.axis_index("s")
  wid = cid * NS + sid
  nbase = sid * NODES_PER_TILE

  pltpu.sync_copy(zeros_hbm, onesbuf)
  for j in range(NODES_PER_TILE // ZB):
    pltpu.sync_copy(onesbuf, dacc.at[pl.ds(nbase + j * ZB, ZB)])
  pltpu.sync_copy(ones_hbm, onesbuf)
  pltpu.sync_copy(col_hbm.at[pl.ds(wid * CHUNKS_PER_TILE, CHUNKS_PER_TILE)],
                  cidx)
  plsc.subcore_barrier()

  def body(j, carry):
    pltpu.sync_copy(onesbuf, dacc.at[cidx.at[j]], add=True)
    return carry
  lax.fori_loop(0, CHUNKS_PER_TILE, body, 0)

  plsc.subcore_barrier()
  for j in range(NODES_PER_TILE // ZB):
    pltpu.sync_copy(dacc.at[pl.ds(nbase + j * ZB, ZB)], onesbuf)
    pltpu.sync_copy(onesbuf, out_hbm.at[cid, pl.ds(nbase + j * ZB, ZB)])


_sc_degree = pl.kernel(
    _sc_degree_body,
    out_type=jax.ShapeDtypeStruct((NC, N_PAD, 16), jnp.float32),
    mesh=_MESH,
    scratch_types=[
        pltpu.VMEM((CHUNKS_PER_TILE, K), jnp.int32),
        pltpu.VMEM((K, 16), jnp.float32),
        pltpu.VMEM_SHARED((N_PAD, 16), jnp.float32),
        pltpu.SemaphoreType.DMA,
    ],
)


def _sc_aggregate_body(row_hbm, col_hbm, g_hbm, zeros_hbm, out_hbm,
                       ridx, cidx, gbuf, acc, sem):
  cid = lax.axis_index("c")
  sid = lax.axis_index("s")
  wid = cid * NS + sid
  nbase = sid * NODES_PER_TILE

  pltpu.sync_copy(zeros_hbm, gbuf)
  for j in range(NODES_PER_TILE // ZB):
    pltpu.sync_copy(gbuf, acc.at[pl.ds(nbase + j * ZB, ZB)])
  pltpu.sync_copy(row_hbm.at[pl.ds(wid * CHUNKS_PER_TILE, CHUNKS_PER_TILE)],
                  ridx)
  pltpu.sync_copy(col_hbm.at[pl.ds(wid * CHUNKS_PER_TILE, CHUNKS_PER_TILE)],
                  cidx)
  plsc.subcore_barrier()

  def body(j, carry):
    pltpu.async_copy(g_hbm.at[ridx.at[j]], gbuf, sem).wait()
    pltpu.sync_copy(gbuf, acc.at[cidx.at[j]], add=True)
    return carry
  lax.fori_loop(0, CHUNKS_PER_TILE, body, 0)

  plsc.subcore_barrier()
  for j in range(NODES_PER_TILE // ZB):
    pltpu.sync_copy(acc.at[pl.ds(nbase + j * ZB, ZB)], gbuf)
    pltpu.sync_copy(gbuf, out_hbm.at[cid, pl.ds(nbase + j * ZB, ZB)])


_sc_aggregate = pl.kernel(
    _sc_aggregate_body,
    out_type=jax.ShapeDtypeStruct((NC, N_PAD, D), jnp.float32),
    mesh=_MESH,
    scratch_types=[
        pltpu.VMEM((CHUNKS_PER_TILE, K), jnp.int32),
        pltpu.VMEM((CHUNKS_PER_TILE, K), jnp.int32),
        pltpu.VMEM((K, D), jnp.float32),
        pltpu.VMEM_SHARED((N_PAD, D), jnp.float32),
        pltpu.SemaphoreType.DMA,
    ],
)


BM = 1000
_GRID = N // BM


def _scale_from_deg(degp_ref):
  d = degp_ref[0, :, 0:1] + degp_ref[1, :, 0:1]
  return jnp.where(d > 0.0, lax.rsqrt(d), 0.0)


def _tc_in_body(degp, x_ref, wi_ref, bi_ref, h_ref, g_ref):
  s = _scale_from_deg(degp)
  z = jnp.dot(x_ref[...], wi_ref[...], preferred_element_type=jnp.float32)
  z = jnp.maximum(z + bi_ref[...], 0.0)
  h_ref[...] = z
  g_ref[...] = z * s


def _tc_layer_body(degp, p_ref, h_ref, wa_ref, wb_ref, h_out, g_out):
  s = _scale_from_deg(degp)
  hn = (p_ref[0] + p_ref[1]) * s
  h = h_ref[...]
  out = jnp.dot(hn, wa_ref[...], preferred_element_type=jnp.float32)
  out = out + jnp.dot(h, wb_ref[...], preferred_element_type=jnp.float32)
  out = jnp.maximum(out + h, 0.0)
  h_out[...] = out
  g_out[...] = out * s


def _tc_last_body(degp, p_ref, h_ref, wa_ref, wb_ref, wc_ref, bc_ref,
                  out_ref):
  s = _scale_from_deg(degp)
  hn = (p_ref[0] + p_ref[1]) * s
  h = h_ref[...]
  out = jnp.dot(hn, wa_ref[...], preferred_element_type=jnp.float32)
  out = out + jnp.dot(h, wb_ref[...], preferred_element_type=jnp.float32)
  out = jnp.maximum(out + h, 0.0)
  out_ref[...] = jnp.dot(out, wc_ref[...],
                         preferred_element_type=jnp.float32) + bc_ref[...]


def _deg_spec():
  return pl.BlockSpec((NC, BM, 16), lambda i: (0, i, 0))


def _p_spec():
  return pl.BlockSpec((NC, BM, D), lambda i: (0, i, 0))


def _mat_spec():
  return pl.BlockSpec((BM, D), lambda i: (i, 0))


def _w_spec():
  return pl.BlockSpec((D, D), lambda i: (0, 0))


_tc_in = pl.pallas_call(
    _tc_in_body,
    grid=(_GRID,),
    in_specs=[_deg_spec(), _mat_spec(), _w_spec(),
              pl.BlockSpec((1, D), lambda i: (0, 0))],
    out_specs=[_mat_spec(), _mat_spec()],
    out_shape=[jax.ShapeDtypeStruct((N, D), jnp.float32)] * 2,
)

_tc_layer = pl.pallas_call(
    _tc_layer_body,
    grid=(_GRID,),
    in_specs=[_deg_spec(), _p_spec(), _mat_spec(), _w_spec(), _w_spec()],
    out_specs=[_mat_spec(), _mat_spec()],
    out_shape=[jax.ShapeDtypeStruct((N, D), jnp.float32)] * 2,
)

_tc_last = pl.pallas_call(
    _tc_last_body,
    grid=(_GRID,),
    in_specs=[_deg_spec(), _p_spec(), _mat_spec(), _w_spec(), _w_spec(),
              _w_spec(), pl.BlockSpec((1, D), lambda i: (0, 0))],
    out_specs=_mat_spec(),
    out_shape=jax.ShapeDtypeStruct((N, D), jnp.float32),
)


@jax.jit
def kernel(x, edge_index, W_in, b_in, W0, W1, W_cls, b_cls):
  row = edge_index[0].astype(jnp.int32)
  col = edge_index[1].astype(jnp.int32)
  pad = E_PAD - E
  row = jnp.concatenate([row, jnp.zeros((pad,), jnp.int32)])
  col = jnp.concatenate([col, jnp.full((pad,), N, jnp.int32)])
  row = row.reshape(E_PAD // K, K)
  col = col.reshape(E_PAD // K, K)
  ones16 = jnp.ones((K, 16), jnp.float32)
  zeros16 = jnp.zeros((ZB, 16), jnp.float32)
  zeros128 = jnp.zeros((ZB, D), jnp.float32)
  assert ZB == K

  rowf = row.reshape(-1); colf = col.reshape(-1)
  deg = jax.ops.segment_sum(jnp.ones((E_PAD,), jnp.float32), colf, num_segments=N_PAD)
  degp = jnp.stack([jnp.broadcast_to(deg[:, None], (N_PAD, 16)),
                    jnp.zeros((N_PAD, 16), jnp.float32)])

  h0, g0 = _tc_in(degp, x, W_in, b_in.reshape(1, D))

  def _agg(g):
    msg = g[jnp.where(rowf < N, rowf, 0)]
    ss = jax.ops.segment_sum(msg, colf, num_segments=N_PAD)
    return jnp.stack([ss, jnp.zeros((N_PAD, D), jnp.float32)])

  p1 = _agg(g0)
  h1, g1 = _tc_layer(degp, p1, h0, W0[:D], W0[D:])

  p2 = _agg(g1)
  wc = jnp.zeros((D, D), jnp.float32).at[:, :C].set(W_cls)
  bc = jnp.zeros((1, D), jnp.float32).at[0, :C].set(b_cls)
  logits = _tc_last(degp, p2, h1, W1[:D], W1[D:], wc, bc)
  return logits[:, :C]


# speedup vs baseline: 2.2729x; 1.2325x over previous
import jax
import jax.numpy as jnp
from jax import lax
from jax.experimental import pallas as pl
from jax.experimental.pallas import tpu as pltpu
from jax.experimental.pallas import tpu_sc as plsc

N = 10000
E = 320000
D = 128
C = 40

NC = 2
NS = 16
NW = NC * NS

E_PAD = 327680
N_ACC = N + 16

GB = 128
SHARD = E_PAD // NW
NBATCH = SHARD // GB

EB = 2048
NCOPY = 4

_MESH = plsc.VectorSubcoreMesh(
    core_axis_name="c", subcore_axis_name="s", num_cores=NC, num_subcores=NS)


def _sc_gather_body(row_hbm, g_hbm, msg_hbm, ridx, gbuf, gbuf2, sem, sem2):
  cid = lax.axis_index("c")
  sid = lax.axis_index("s")
  wid = cid * NS + sid
  ebase = wid * SHARD

  pltpu.sync_copy(row_hbm.at[pl.ds(ebase, SHARD)], ridx)

  cp = pltpu.async_copy(g_hbm.at[ridx.at[pl.ds(0, GB)]], gbuf, sem)
  for b in range(NBATCH):
    buf_cur = gbuf if b % 2 == 0 else gbuf2
    buf_nxt = gbuf2 if b % 2 == 0 else gbuf
    cp.wait()
    if b + 1 < NBATCH:
      cp = pltpu.async_copy(
          g_hbm.at[ridx.at[pl.ds((b + 1) * GB, GB)]], buf_nxt, sem)
    wr = pltpu.async_copy(
        buf_cur, msg_hbm.at[pl.ds(ebase + b * GB, GB)], sem2)
    if b + 1 == NBATCH:
      wr.wait()
    else:
      wr.wait()


_sc_gather = pl.kernel(
    _sc_gather_body,
    out_type=jax.ShapeDtypeStruct((E_PAD, D), jnp.float32),
    mesh=_MESH,
    scratch_types=[
        pltpu.VMEM((SHARD,), jnp.int32),
        pltpu.VMEM((GB, D), jnp.float32),
        pltpu.VMEM((GB, D), jnp.float32),
        pltpu.SemaphoreType.DMA,
        pltpu.SemaphoreType.DMA,
    ],
)


_SGRID = E_PAD // EB


def _tc_scatter_body(col_ref, msg_ref, out_ref, a1, a2, a3):
  i = pl.program_id(0)

  @pl.when(i == 0)
  def _zero():
    z = jnp.zeros((N_ACC, D), jnp.float32)
    out_ref[...] = z
    a1[...] = z
    a2[...] = z
    a3[...] = z

  accs = (out_ref, a1, a2, a3)

  def body(j, carry):
    for u in range(NCOPY):
      e = j * NCOPY + u
      c = col_ref[e]
      t = accs[u]
      t[pl.ds(c, 1), :] = t[pl.ds(c, 1), :] + msg_ref[pl.ds(e, 1), :]
    return carry
  lax.fori_loop(0, EB // NCOPY, body, 0)

  @pl.when(i == _SGRID - 1)
  def _reduce():
    out_ref[...] = out_ref[...] + a1[...] + a2[...] + a3[...]


_tc_scatter = pl.pallas_call(
    _tc_scatter_body,
    grid=(_SGRID,),
    in_specs=[
        pl.BlockSpec((EB,), lambda i: (i,), memory_space=pltpu.SMEM),
        pl.BlockSpec((EB, D), lambda i: (i, 0)),
    ],
    out_specs=pl.BlockSpec((N_ACC, D), lambda i: (0, 0)),
    out_shape=jax.ShapeDtypeStruct((N_ACC, D), jnp.float32),
    scratch_shapes=[pltpu.VMEM((N_ACC, D), jnp.float32)] * 3,
)


def _tc_degree_body(col_ref, out_ref, a1, a2, a3):
  i = pl.program_id(0)

  @pl.when(i == 0)
  def _zero():
    z = jnp.zeros((N_ACC, D), jnp.float32)
    out_ref[...] = z
    a1[...] = z
    a2[...] = z
    a3[...] = z

  accs = (out_ref, a1, a2, a3)
  one = jnp.ones((1, D), jnp.float32)

  def body(j, carry):
    for u in range(NCOPY):
      e = j * NCOPY + u
      c = col_ref[e]
      t = accs[u]
      t[pl.ds(c, 1), :] = t[pl.ds(c, 1), :] + one
    return carry
  lax.fori_loop(0, EB // NCOPY, body, 0)

  @pl.when(i == _SGRID - 1)
  def _reduce():
    out_ref[...] = out_ref[...] + a1[...] + a2[...] + a3[...]


_tc_degree = pl.pallas_call(
    _tc_degree_body,
    grid=(_SGRID,),
    in_specs=[pl.BlockSpec((EB,), lambda i: (i,), memory_space=pltpu.SMEM)],
    out_specs=pl.BlockSpec((N_ACC, D), lambda i: (0, 0)),
    out_shape=jax.ShapeDtypeStruct((N_ACC, D), jnp.float32),
    scratch_shapes=[pltpu.VMEM((N_ACC, D), jnp.float32)] * 3,
)


BM = 1000
_GRID = N // BM


def _scale_from_deg(deg_ref):
  d = deg_ref[:, 0:1]
  return jnp.where(d > 0.0, lax.rsqrt(d), 0.0)


def _tc_in_body(deg, x_ref, wi_ref, bi_ref, h_ref, g_ref):
  s = _scale_from_deg(deg)
  z = jnp.dot(x_ref[...], wi_ref[...], preferred_element_type=jnp.float32)
  z = jnp.maximum(z + bi_ref[...], 0.0)
  h_ref[...] = z
  g_ref[...] = z * s


def _tc_layer_body(deg, p_ref, h_ref, wa_ref, wb_ref, h_out, g_out):
  s = _scale_from_deg(deg)
  hn = p_ref[...] * s
  h = h_ref[...]
  out = jnp.dot(hn, wa_ref[...], preferred_element_type=jnp.float32)
  out = out + jnp.dot(h, wb_ref[...], preferred_element_type=jnp.float32)
  out = jnp.maximum(out + h, 0.0)
  h_out[...] = out
  g_out[...] = out * s


def _tc_last_body(deg, p_ref, h_ref, wa_ref, wb_ref, wc_ref, bc_ref,
                  out_ref):
  s = _scale_from_deg(deg)
  hn = p_ref[...] * s
  h = h_ref[...]
  out = jnp.dot(hn, wa_ref[...], preferred_element_type=jnp.float32)
  out = out + jnp.dot(h, wb_ref[...], preferred_element_type=jnp.float32)
  out = jnp.maximum(out + h, 0.0)
  out_ref[...] = jnp.dot(out, wc_ref[...],
                         preferred_element_type=jnp.float32) + bc_ref[...]


def _mat_spec():
  return pl.BlockSpec((BM, D), lambda i: (i, 0))


def _w_spec():
  return pl.BlockSpec((D, D), lambda i: (0, 0))


_tc_in = pl.pallas_call(
    _tc_in_body,
    grid=(_GRID,),
    in_specs=[_mat_spec(), _mat_spec(), _w_spec(),
              pl.BlockSpec((1, D), lambda i: (0, 0))],
    out_specs=[_mat_spec(), _mat_spec()],
    out_shape=[jax.ShapeDtypeStruct((N, D), jnp.float32)] * 2,
)

_tc_layer = pl.pallas_call(
    _tc_layer_body,
    grid=(_GRID,),
    in_specs=[_mat_spec(), _mat_spec(), _mat_spec(), _w_spec(), _w_spec()],
    out_specs=[_mat_spec(), _mat_spec()],
    out_shape=[jax.ShapeDtypeStruct((N, D), jnp.float32)] * 2,
)

_tc_last = pl.pallas_call(
    _tc_last_body,
    grid=(_GRID,),
    in_specs=[_mat_spec(), _mat_spec(), _mat_spec(), _w_spec(), _w_spec(),
              _w_spec(), pl.BlockSpec((1, D), lambda i: (0, 0))],
    out_specs=_mat_spec(),
    out_shape=jax.ShapeDtypeStruct((N, D), jnp.float32),
)


@jax.jit
def kernel(x, edge_index, W_in, b_in, W0, W1, W_cls, b_cls):
  row = edge_index[0].astype(jnp.int32)
  col = edge_index[1].astype(jnp.int32)
  pad = E_PAD - E
  row = jnp.concatenate([row, jnp.zeros((pad,), jnp.int32)])
  col = jnp.concatenate([col, jnp.full((pad,), N, jnp.int32)])

  deg = _tc_degree(col)[:N]

  h0, g0 = _tc_in(deg, x, W_in, b_in.reshape(1, D))

  p1 = _tc_scatter(col, _sc_gather(row, g0))[:N]
  h1, g1 = _tc_layer(deg, p1, h0, W0[:D], W0[D:])

  p2 = _tc_scatter(col, _sc_gather(row, g1))[:N]
  wc = jnp.zeros((D, D), jnp.float32).at[:, :C].set(W_cls)
  bc = jnp.zeros((1, D), jnp.float32).at[0, :C].set(b_cls)
  logits = _tc_last(deg, p2, h1, W1[:D], W1[D:], wc, bc)
  return logits[:, :C]


# speedup vs baseline: 2.9090x; 1.2798x over previous
import jax
import jax.numpy as jnp
from jax import lax
from jax.experimental import pallas as pl
from jax.experimental.pallas import tpu as pltpu
from jax.experimental.pallas import tpu_sc as plsc

N = 10000
E = 320000
D = 128
C = 40

NC = 2
NS = 16
NW = NC * NS

E_PAD = 327680
N_ACC = N + 16

GB = 128
SHARD = E_PAD // NW
NBATCH = SHARD // GB

EB = 4096
NCOPY = 8

_MESH = plsc.VectorSubcoreMesh(
    core_axis_name="c", subcore_axis_name="s", num_cores=NC, num_subcores=NS)


def _sc_gather_body(row_hbm, g_hbm, msg_hbm, ridx, gbuf, gbuf2, sem, sem2):
  cid = lax.axis_index("c")
  sid = lax.axis_index("s")
  wid = cid * NS + sid
  ebase = wid * SHARD

  pltpu.sync_copy(row_hbm.at[pl.ds(ebase, SHARD)], ridx)

  cp = pltpu.async_copy(g_hbm.at[ridx.at[pl.ds(0, GB)]], gbuf, sem)
  wr = None
  for b in range(NBATCH):
    buf_cur = gbuf if b % 2 == 0 else gbuf2
    buf_nxt = gbuf2 if b % 2 == 0 else gbuf
    cp.wait()
    if wr is not None:
      wr.wait()
    if b + 1 < NBATCH:
      cp = pltpu.async_copy(
          g_hbm.at[ridx.at[pl.ds((b + 1) * GB, GB)]], buf_nxt, sem)
    wr = pltpu.async_copy(
        buf_cur, msg_hbm.at[pl.ds(ebase + b * GB, GB)], sem2)
  wr.wait()


_sc_gather = pl.kernel(
    _sc_gather_body,
    out_type=jax.ShapeDtypeStruct((E_PAD, D), jnp.float32),
    mesh=_MESH,
    scratch_types=[
        pltpu.VMEM((SHARD,), jnp.int32),
        pltpu.VMEM((GB, D), jnp.float32),
        pltpu.VMEM((GB, D), jnp.float32),
        pltpu.SemaphoreType.DMA,
        pltpu.SemaphoreType.DMA,
    ],
)


_SGRID = E_PAD // EB


def _tc_scatter_body(col_ref, msg_ref, out_ref, a1, a2, a3, a4, a5, a6, a7):
  i = pl.program_id(0)
  accs = (out_ref, a1, a2, a3, a4, a5, a6, a7)

  @pl.when(i == 0)
  def _zero():
    z = jnp.zeros((N_ACC, D), jnp.float32)
    for t in accs:
      t[...] = z

  def body(j, carry):
    for u in range(NCOPY):
      e = j * NCOPY + u
      c = col_ref[e]
      t = accs[u]
      t[pl.ds(c, 1), :] = t[pl.ds(c, 1), :] + msg_ref[pl.ds(e, 1), :]
    return carry
  lax.fori_loop(0, EB // NCOPY, body, 0)

  @pl.when(i == _SGRID - 1)
  def _reduce():
    out_ref[...] = (((out_ref[...] + a1[...]) + (a2[...] + a3[...]))
                    + ((a4[...] + a5[...]) + (a6[...] + a7[...])))


_tc_scatter = pl.pallas_call(
    _tc_scatter_body,
    grid=(_SGRID,),
    in_specs=[
        pl.BlockSpec((EB,), lambda i: (i,), memory_space=pltpu.SMEM),
        pl.BlockSpec((EB, D), lambda i: (i, 0)),
    ],
    out_specs=pl.BlockSpec((N_ACC, D), lambda i: (0, 0)),
    out_shape=jax.ShapeDtypeStruct((N_ACC, D), jnp.float32),
    scratch_shapes=[pltpu.VMEM((N_ACC, D), jnp.float32)] * 7,
)


def _tc_degree_body(col_ref, out_ref, a1, a2, a3, a4, a5, a6, a7):
  i = pl.program_id(0)
  accs = (out_ref, a1, a2, a3, a4, a5, a6, a7)

  @pl.when(i == 0)
  def _zero():
    z = jnp.zeros((N_ACC, D), jnp.float32)
    for t in accs:
      t[...] = z

  one = jnp.ones((1, D), jnp.float32)

  def body(j, carry):
    for u in range(NCOPY):
      e = j * NCOPY + u
      c = col_ref[e]
      t = accs[u]
      t[pl.ds(c, 1), :] = t[pl.ds(c, 1), :] + one
    return carry
  lax.fori_loop(0, EB // NCOPY, body, 0)

  @pl.when(i == _SGRID - 1)
  def _reduce():
    out_ref[...] = (((out_ref[...] + a1[...]) + (a2[...] + a3[...]))
                    + ((a4[...] + a5[...]) + (a6[...] + a7[...])))


_tc_degree = pl.pallas_call(
    _tc_degree_body,
    grid=(_SGRID,),
    in_specs=[pl.BlockSpec((EB,), lambda i: (i,), memory_space=pltpu.SMEM)],
    out_specs=pl.BlockSpec((N_ACC, D), lambda i: (0, 0)),
    out_shape=jax.ShapeDtypeStruct((N_ACC, D), jnp.float32),
    scratch_shapes=[pltpu.VMEM((N_ACC, D), jnp.float32)] * 7,
)


BM = 1000
_GRID = N // BM


def _scale_from_deg(deg_ref):
  d = deg_ref[:, 0:1]
  return jnp.where(d > 0.0, lax.rsqrt(d), 0.0)


def _tc_in_body(deg, x_ref, wi_ref, bi_ref, h_ref, g_ref):
  s = _scale_from_deg(deg)
  z = jnp.dot(x_ref[...], wi_ref[...], preferred_element_type=jnp.float32)
  z = jnp.maximum(z + bi_ref[...], 0.0)
  h_ref[...] = z
  g_ref[...] = z * s


def _tc_layer_body(deg, p_ref, h_ref, wa_ref, wb_ref, h_out, g_out):
  s = _scale_from_deg(deg)
  hn = p_ref[...] * s
  h = h_ref[...]
  out = jnp.dot(hn, wa_ref[...], preferred_element_type=jnp.float32)
  out = out + jnp.dot(h, wb_ref[...], preferred_element_type=jnp.float32)
  out = jnp.maximum(out + h, 0.0)
  h_out[...] = out
  g_out[...] = out * s


def _tc_last_body(deg, p_ref, h_ref, wa_ref, wb_ref, wc_ref, bc_ref,
                  out_ref):
  s = _scale_from_deg(deg)
  hn = p_ref[...] * s
  h = h_ref[...]
  out = jnp.dot(hn, wa_ref[...], preferred_element_type=jnp.float32)
  out = out + jnp.dot(h, wb_ref[...], preferred_element_type=jnp.float32)
  out = jnp.maximum(out + h, 0.0)
  out_ref[...] = jnp.dot(out, wc_ref[...],
                         preferred_element_type=jnp.float32) + bc_ref[...]


def _mat_spec():
  return pl.BlockSpec((BM, D), lambda i: (i, 0))


def _w_spec():
  return pl.BlockSpec((D, D), lambda i: (0, 0))


_tc_in = pl.pallas_call(
    _tc_in_body,
    grid=(_GRID,),
    in_specs=[_mat_spec(), _mat_spec(), _w_spec(),
              pl.BlockSpec((1, D), lambda i: (0, 0))],
    out_specs=[_mat_spec(), _mat_spec()],
    out_shape=[jax.ShapeDtypeStruct((N, D), jnp.float32)] * 2,
)

_tc_layer = pl.pallas_call(
    _tc_layer_body,
    grid=(_GRID,),
    in_specs=[_mat_spec(), _mat_spec(), _mat_spec(), _w_spec(), _w_spec()],
    out_specs=[_mat_spec(), _mat_spec()],
    out_shape=[jax.ShapeDtypeStruct((N, D), jnp.float32)] * 2,
)

_tc_last = pl.pallas_call(
    _tc_last_body,
    grid=(_GRID,),
    in_specs=[_mat_spec(), _mat_spec(), _mat_spec(), _w_spec(), _w_spec(),
              _w_spec(), pl.BlockSpec((1, D), lambda i: (0, 0))],
    out_specs=_mat_spec(),
    out_shape=jax.ShapeDtypeStruct((N, D), jnp.float32),
)


@jax.jit
def kernel(x, edge_index, W_in, b_in, W0, W1, W_cls, b_cls):
  row = edge_index[0].astype(jnp.int32)
  col = edge_index[1].astype(jnp.int32)
  pad = E_PAD - E
  row = jnp.concatenate([row, jnp.zeros((pad,), jnp.int32)])
  col = jnp.concatenate([col, jnp.full((pad,), N, jnp.int32)])

  deg = _tc_degree(col)[:N]

  h0, g0 = _tc_in(deg, x, W_in, b_in.reshape(1, D))

  p1 = _tc_scatter(col, _sc_gather(row, g0))[:N]
  h1, g1 = _tc_layer(deg, p1, h0, W0[:D], W0[D:])

  p2 = _tc_scatter(col, _sc_gather(row, g1))[:N]
  wc = jnp.zeros((D, D), jnp.float32).at[:, :C].set(W_cls)
  bc = jnp.zeros((1, D), jnp.float32).at[0, :C].set(b_cls)
  logits = _tc_last(deg, p2, h1, W1[:D], W1[D:], wc, bc)
  return logits[:, :C]


# speedup vs baseline: 2.9723x; 1.0218x over previous
import jax
import jax.numpy as jnp
from jax import lax
from jax.experimental import pallas as pl
from jax.experimental.pallas import tpu as pltpu
from jax.experimental.pallas import tpu_sc as plsc

N = 10000
E = 320000
D = 128
C = 40

NC = 2
NS = 16
NW = NC * NS

E_PAD = 327680
N_ACC = N + 16

GB = 128
SHARD_S = E_PAD // (4 * NS)
SHARD_F = 3 * E_PAD // (4 * NS)
SLOW_CID = 0

EB = 4096
NCOPY = 8

_MESH = plsc.VectorSubcoreMesh(
    core_axis_name="c", subcore_axis_name="s", num_cores=NC, num_subcores=NS)


def _sc_gather_body(row_hbm, g_hbm, msg_hbm, ridx, gbuf, gbuf2, gbuf3,
                    sem, sem2):
  cid = lax.axis_index("c")
  sid = lax.axis_index("s")
  bufs = (gbuf, gbuf2, gbuf3)

  def pipeline(ebase, shard):
    nbatch = shard // GB
    pltpu.sync_copy(row_hbm.at[pl.ds(ebase, shard)],
                    ridx.at[pl.ds(0, shard)])

    def start_gather(b):
      return pltpu.async_copy(
          g_hbm.at[ridx.at[pl.ds(b * GB, GB)]], bufs[b % 3], sem)

    cps = [start_gather(0), start_gather(1)]
    wr = None
    for b in range(nbatch):
      cps[b % 2].wait()
      if wr is not None:
        wr.wait()
      if b + 2 < nbatch:
        cps[b % 2] = start_gather(b + 2)
      wr = pltpu.async_copy(
          bufs[b % 3], msg_hbm.at[pl.ds(ebase + b * GB, GB)], sem2)
    wr.wait()

  @pl.when(cid == SLOW_CID)
  def _slow():
    pipeline(sid * SHARD_S, SHARD_S)

  @pl.when(cid != SLOW_CID)
  def _fast():
    pipeline(NS * SHARD_S + sid * SHARD_F, SHARD_F)


_sc_gather = pl.kernel(
    _sc_gather_body,
    out_type=jax.ShapeDtypeStruct((E_PAD, D), jnp.float32),
    mesh=_MESH,
    scratch_types=[
        pltpu.VMEM((SHARD_F,), jnp.int32),
        pltpu.VMEM((GB, D), jnp.float32),
        pltpu.VMEM((GB, D), jnp.float32),
        pltpu.VMEM((GB, D), jnp.float32),
        pltpu.SemaphoreType.DMA,
        pltpu.SemaphoreType.DMA,
    ],
)


_SGRID = E_PAD // EB


def _tc_scatter_body(col_ref, msg_ref, out_ref, a1, a2, a3, a4, a5, a6, a7):
  i = pl.program_id(0)
  accs = (out_ref, a1, a2, a3, a4, a5, a6, a7)

  @pl.when(i == 0)
  def _zero():
    z = jnp.zeros((N_ACC, D), jnp.float32)
    for t in accs:
      t[...] = z

  def body(j, carry):
    for u in range(NCOPY):
      e = j * NCOPY + u
      c = col_ref[e]
      t = accs[u]
      t[pl.ds(c, 1), :] = t[pl.ds(c, 1), :] + msg_ref[pl.ds(e, 1), :]
    return carry
  lax.fori_loop(0, EB // NCOPY, body, 0)

  @pl.when(i == _SGRID - 1)
  def _reduce():
    out_ref[...] = (((out_ref[...] + a1[...]) + (a2[...] + a3[...]))
                    + ((a4[...] + a5[...]) + (a6[...] + a7[...])))


_tc_scatter = pl.pallas_call(
    _tc_scatter_body,
    grid=(_SGRID,),
    in_specs=[
        pl.BlockSpec((EB,), lambda i: (i,), memory_space=pltpu.SMEM),
        pl.BlockSpec((EB, D), lambda i: (i, 0)),
    ],
    out_specs=pl.BlockSpec((N_ACC, D), lambda i: (0, 0)),
    out_shape=jax.ShapeDtypeStruct((N_ACC, D), jnp.float32),
    scratch_shapes=[pltpu.VMEM((N_ACC, D), jnp.float32)] * 7,
)


def _tc_degree_body(col_ref, out_ref, a1, a2, a3, a4, a5, a6, a7):
  i = pl.program_id(0)
  accs = (out_ref, a1, a2, a3, a4, a5, a6, a7)

  @pl.when(i == 0)
  def _zero():
    z = jnp.zeros((N_ACC, D), jnp.float32)
    for t in accs:
      t[...] = z

  one = jnp.ones((1, D), jnp.float32)

  def body(j, carry):
    for u in range(NCOPY):
      e = j * NCOPY + u
      c = col_ref[e]
      t = accs[u]
      t[pl.ds(c, 1), :] = t[pl.ds(c, 1), :] + one
    return carry
  lax.fori_loop(0, EB // NCOPY, body, 0)

  @pl.when(i == _SGRID - 1)
  def _reduce():
    out_ref[...] = (((out_ref[...] + a1[...]) + (a2[...] + a3[...]))
                    + ((a4[...] + a5[...]) + (a6[...] + a7[...])))


_tc_degree = pl.pallas_call(
    _tc_degree_body,
    grid=(_SGRID,),
    in_specs=[pl.BlockSpec((EB,), lambda i: (i,), memory_space=pltpu.SMEM)],
    out_specs=pl.BlockSpec((N_ACC, D), lambda i: (0, 0)),
    out_shape=jax.ShapeDtypeStruct((N_ACC, D), jnp.float32),
    scratch_shapes=[pltpu.VMEM((N_ACC, D), jnp.float32)] * 7,
)


BM = 1000
_GRID = N // BM


def _scale_from_deg(deg_ref):
  d = deg_ref[:, 0:1]
  return jnp.where(d > 0.0, lax.rsqrt(d), 0.0)


def _tc_in_body(deg, x_ref, wi_ref, bi_ref, h_ref, g_ref):
  s = _scale_from_deg(deg)
  z = jnp.dot(x_ref[...], wi_ref[...], preferred_element_type=jnp.float32)
  z = jnp.maximum(z + bi_ref[...], 0.0)
  h_ref[...] = z
  g_ref[...] = z * s


def _tc_layer_body(deg, p_ref, h_ref, wa_ref, wb_ref, h_out, g_out):
  s = _scale_from_deg(deg)
  hn = p_ref[...] * s
  h = h_ref[...]
  out = jnp.dot(hn, wa_ref[...], preferred_element_type=jnp.float32)
  out = out + jnp.dot(h, wb_ref[...], preferred_element_type=jnp.float32)
  out = jnp.maximum(out + h, 0.0)
  h_out[...] = out
  g_out[...] = out * s


def _tc_last_body(deg, p_ref, h_ref, wa_ref, wb_ref, wc_ref, bc_ref,
                  out_ref):
  s = _scale_from_deg(deg)
  hn = p_ref[...] * s
  h = h_ref[...]
  out = jnp.dot(hn, wa_ref[...], preferred_element_type=jnp.float32)
  out = out + jnp.dot(h, wb_ref[...], preferred_element_type=jnp.float32)
  out = jnp.maximum(out + h, 0.0)
  out_ref[...] = jnp.dot(out, wc_ref[...],
                         preferred_element_type=jnp.float32) + bc_ref[...]


def _mat_spec():
  return pl.BlockSpec((BM, D), lambda i: (i, 0))


def _w_spec():
  return pl.BlockSpec((D, D), lambda i: (0, 0))


_tc_in = pl.pallas_call(
    _tc_in_body,
    grid=(_GRID,),
    in_specs=[_mat_spec(), _mat_spec(), _w_spec(),
              pl.BlockSpec((1, D), lambda i: (0, 0))],
    out_specs=[_mat_spec(), _mat_spec()],
    out_shape=[jax.ShapeDtypeStruct((N, D), jnp.float32)] * 2,
)

_tc_layer = pl.pallas_call(
    _tc_layer_body,
    grid=(_GRID,),
    in_specs=[_mat_spec(), _mat_spec(), _mat_spec(), _w_spec(), _w_spec()],
    out_specs=[_mat_spec(), _mat_spec()],
    out_shape=[jax.ShapeDtypeStruct((N, D), jnp.float32)] * 2,
)

_tc_last = pl.pallas_call(
    _tc_last_body,
    grid=(_GRID,),
    in_specs=[_mat_spec(), _mat_spec(), _mat_spec(), _w_spec(), _w_spec(),
              _w_spec(), pl.BlockSpec((1, D), lambda i: (0, 0))],
    out_specs=_mat_spec(),
    out_shape=jax.ShapeDtypeStruct((N, D), jnp.float32),
)


@jax.jit
def kernel(x, edge_index, W_in, b_in, W0, W1, W_cls, b_cls):
  row = edge_index[0].astype(jnp.int32)
  col = edge_index[1].astype(jnp.int32)
  pad = E_PAD - E
  row = jnp.concatenate([row, jnp.zeros((pad,), jnp.int32)])
  col = jnp.concatenate([col, jnp.full((pad,), N, jnp.int32)])

  deg = _tc_degree(col)[:N]

  h0, g0 = _tc_in(deg, x, W_in, b_in.reshape(1, D))

  p1 = _tc_scatter(col, _sc_gather(row, g0))[:N]
  h1, g1 = _tc_layer(deg, p1, h0, W0[:D], W0[D:])

  p2 = _tc_scatter(col, _sc_gather(row, g1))[:N]
  wc = jnp.zeros((D, D), jnp.float32).at[:, :C].set(W_cls)
  bc = jnp.zeros((1, D), jnp.float32).at[0, :C].set(b_cls)
  logits = _tc_last(deg, p2, h1, W1[:D], W1[D:], wc, bc)
  return logits[:, :C]
